# baseline (speedup 1.0000x reference)
import functools

import jax
import jax.numpy as jnp
from jax import lax
from jax.experimental import pallas as pl
from jax.experimental.pallas import tpu as pltpu
from jax.experimental.pallas import tpu_sc as plsc

_N_ROWS = 100000
_D = 64
_B = 16384

_CHUNK = 128
_NW = 32
_K = _B // (_CHUNK * _NW)

_mesh = plsc.VectorSubcoreMesh(core_axis_name="c", subcore_axis_name="s")


@functools.partial(
    pl.kernel,
    mesh=_mesh,
    compiler_params=pltpu.CompilerParams(use_tc_tiling_on_sc=False),
    out_type=jax.ShapeDtypeStruct((_B // _CHUNK, _CHUNK, _D), jnp.float32),
    scratch_types=[
        pltpu.VMEM((_K, _CHUNK), jnp.int32),
        pltpu.VMEM((_K, _CHUNK, _D), jnp.float32),
        pltpu.SemaphoreType.DMA,
    ],
)
def _gather_kernel(idx_hbm, table_hbm, out_hbm, idx_v, rows_v, sem):
    wid = lax.axis_index("s") * 2 + lax.axis_index("c")
    base = wid * _K
    pltpu.sync_copy(idx_hbm.at[pl.ds(base, _K)], idx_v)
    copies = [
        pltpu.async_copy(table_hbm.at[idx_v.at[c]], rows_v.at[c], sem)
        for c in range(_K)
    ]
    for cp in copies:
        cp.wait()
    pltpu.sync_copy(rows_v, out_hbm.at[pl.ds(base, _K)])


def kernel(x, emb_weight):
    idx = x.astype(jnp.int32).reshape(_B // _CHUNK, _CHUNK)
    out = _gather_kernel(idx, emb_weight)
    return out.reshape(_B, _D)


# speedup vs baseline: 1.0001x; 1.0001x over previous
import functools

import jax
import jax.numpy as jnp
from jax import lax
from jax.experimental import pallas as pl
from jax.experimental.pallas import tpu as pltpu
from jax.experimental.pallas import tpu_sc as plsc

_N_ROWS = 100000
_D = 64
_B = 16384

_CHUNK = 128
_NW = 32
_BPW = _B // _NW
_K = _BPW // _CHUNK

_mesh = plsc.VectorSubcoreMesh(core_axis_name="c", subcore_axis_name="s")


@functools.partial(
    pl.kernel,
    mesh=_mesh,
    compiler_params=pltpu.CompilerParams(use_tc_tiling_on_sc=False),
    out_type=jax.ShapeDtypeStruct((_B, _D), jnp.float32),
    scratch_types=[
        pltpu.VMEM((_BPW,), jnp.int32),
        pltpu.VMEM((_BPW, _D), jnp.float32),
        pltpu.SemaphoreType.DMA,
    ],
)
def _gather_kernel(idx_hbm, table_hbm, out_hbm, idx_v, rows_v, sem):
    wid = lax.axis_index("s") * 2 + lax.axis_index("c")
    base = wid * _BPW
    pltpu.sync_copy(idx_hbm.at[pl.ds(base, _BPW)], idx_v)
    copies = [
        pltpu.async_copy(
            table_hbm.at[idx_v.at[pl.ds(c * _CHUNK, _CHUNK)]],
            rows_v.at[pl.ds(c * _CHUNK, _CHUNK)],
            sem,
        )
        for c in range(_K)
    ]
    for cp in copies:
        cp.wait()
    pltpu.sync_copy(rows_v, out_hbm.at[pl.ds(base, _BPW)])


def kernel(x, emb_weight):
    return _gather_kernel(x.astype(jnp.int32), emb_weight)


# speedup vs baseline: 1.8338x; 1.8336x over previous
import functools

import jax
import jax.numpy as jnp
from jax import lax
from jax.experimental import pallas as pl
from jax.experimental.pallas import tpu as pltpu
from jax.experimental.pallas import tpu_sc as plsc

_V = 100000
_D = 64
_B = 16384
_NS = 16
_NB = 15
_RPC = 32

_mesh = plsc.VectorSubcoreMesh(core_axis_name="c", subcore_axis_name="s")


@functools.partial(
    pl.kernel,
    mesh=_mesh,
    compiler_params=pltpu.CompilerParams(use_tc_tiling_on_sc=True),
    out_type=jax.ShapeDtypeStruct((_D, _B), jnp.float32),
    scratch_types=(
        [pltpu.VMEM_SHARED((_V,), jnp.float32) for _ in range(_NB)]
        + [
            pltpu.VMEM((_B,), jnp.int32),
            pltpu.VMEM((_B,), jnp.float32),
            pltpu.SemaphoreType.DMA,
        ]
    ),
)
def _gather_kernel(wt_hbm, idx_hbm, out_hbm, *refs):
    bufs = refs[:_NB]
    idx_v, o_v, sem = refs[_NB:]
    c = lax.axis_index("c")
    s = lax.axis_index("s")
    pltpu.sync_copy(idx_hbm, idx_v)
    k = 0
    while k < _RPC:
        n = min(_NB, _RPC - k)
        for i in range(n):
            @pl.when(s == i)
            def _(k=k, i=i):
                d = c * _RPC + k + i
                pltpu.sync_copy(wt_hbm.at[d], bufs[i])
                pltpu.async_copy(bufs[i].at[idx_v], o_v, sem).wait()
                pltpu.sync_copy(o_v, out_hbm.at[d])
        k += n


def kernel(x, emb_weight):
    out_t = _gather_kernel(emb_weight.T, x.astype(jnp.int32))
    return out_t.T
